# baseline (speedup 1.0000x reference)
import functools

import jax
import jax.numpy as jnp
from jax import lax
from jax.experimental import pallas as pl
from jax.experimental.pallas import tpu as pltpu
from jax.experimental.pallas import tpu_sc as plsc

_NBUF = 4
_GSZ = 128


@functools.partial(jax.jit, static_argnums=(2, 3))
def _embed_sc(xpad, weight, nc, ns):
    num, dim = weight.shape
    nw = nc * ns
    n_lookups = xpad.shape[0] * xpad.shape[1]
    lpw = n_lookups // nw
    n_groups = lpw // _GSZ
    rows_per_w = xpad.shape[0] // nw
    mesh = plsc.VectorSubcoreMesh(core_axis_name="c", subcore_axis_name="s")

    @functools.partial(
        pl.kernel,
        out_type=jax.ShapeDtypeStruct((n_lookups, dim), jnp.float32),
        mesh=mesh,
        scratch_types=[
            pltpu.VMEM((rows_per_w, _GSZ), jnp.int32),
            pltpu.VMEM((_NBUF, _GSZ, dim), jnp.float32),
            pltpu.SemaphoreType.DMA((_NBUF,)),
            pltpu.SemaphoreType.DMA((_NBUF,)),
        ],
        compiler_params=pltpu.CompilerParams(use_tc_tiling_on_sc=False),
    )
    def k(x_hbm, w_hbm, out_hbm, idx_v, rows_v, gsem, wsem):
        wid = lax.axis_index("s") * nc + lax.axis_index("c")
        r0 = wid * rows_per_w
        pltpu.sync_copy(x_hbm.at[pl.ds(r0, rows_per_w)], idx_v)
        o0 = wid * lpw

        def start_gather(g, b):
            pltpu.make_async_copy(
                w_hbm.at[idx_v.at[g]], rows_v.at[b], gsem.at[b]
            ).start()

        def wait_gather(g, b):
            pltpu.make_async_copy(
                w_hbm.at[idx_v.at[g]], rows_v.at[b], gsem.at[b]
            ).wait()

        def write_copy(g, b):
            return pltpu.make_async_copy(
                rows_v.at[b],
                out_hbm.at[pl.ds(o0 + g * _GSZ, _GSZ)],
                wsem.at[b],
            )

        for b in range(_NBUF):
            start_gather(b, b)

        n_outer = (n_groups + _NBUF - 1) // _NBUF

        def body(j, carry):
            for b in range(_NBUF):
                g = j * _NBUF + b

                @pl.when(g < n_groups)
                def _():
                    wait_gather(g, b)
                    write_copy(g, b).start()

                    @pl.when(g + _NBUF < n_groups)
                    def _():
                        write_copy(g, b).wait()
                        start_gather(g + _NBUF, b)

            return carry

        lax.fori_loop(0, n_outer, body, 0)
        for b in range(_NBUF):
            write_copy(0, b).wait()

    return k(xpad, weight)


def kernel(x, weight):
    b, f = x.shape
    dim = weight.shape[1]
    info = plsc.get_sparse_core_info()
    nc, ns = info.num_cores, info.num_subcores
    fpad = 32
    xp = jnp.pad(x, ((0, 0), (0, fpad - f))).reshape(b * fpad // 128, 128)
    out = _embed_sc(xp, weight, nc, ns)
    return out.reshape(b, fpad, dim)[:, :f, :]


# speedup vs baseline: 3.8405x; 3.8405x over previous
import functools

import jax
import jax.numpy as jnp
from jax import lax
from jax.experimental import pallas as pl
from jax.experimental.pallas import tpu as pltpu
from jax.experimental.pallas import tpu_sc as plsc

_NBUF = 4
_GSZ = 128


@functools.partial(jax.jit, static_argnums=(2, 3))
def _embed_sc(xpad, weight, nc, ns):
    num, dim = weight.shape
    nw = nc * ns
    n_lookups = xpad.shape[0] * xpad.shape[1]
    lpw = n_lookups // nw
    n_groups = lpw // _GSZ
    rows_per_w = xpad.shape[0] // nw
    mesh = plsc.VectorSubcoreMesh(core_axis_name="c", subcore_axis_name="s")

    @functools.partial(
        pl.kernel,
        out_type=jax.ShapeDtypeStruct((n_lookups, dim), jnp.float32),
        mesh=mesh,
        scratch_types=[
            pltpu.VMEM((rows_per_w, _GSZ), jnp.int32),
            pltpu.VMEM((_NBUF, _GSZ, dim), jnp.float32),
            pltpu.SemaphoreType.DMA((_NBUF,)),
            pltpu.SemaphoreType.DMA((_NBUF,)),
        ],
        compiler_params=pltpu.CompilerParams(use_tc_tiling_on_sc=False),
    )
    def k(x_hbm, w_hbm, out_hbm, idx_v, rows_v, gsem, wsem):
        wid = lax.axis_index("s") * nc + lax.axis_index("c")
        r0 = wid * rows_per_w
        pltpu.sync_copy(x_hbm.at[pl.ds(r0, rows_per_w)], idx_v)
        o0 = wid * lpw

        def start_gather(g, b):
            pltpu.make_async_copy(
                w_hbm.at[idx_v.at[g]], rows_v.at[b], gsem.at[b]
            ).start()

        def wait_gather(g, b):
            pltpu.make_async_copy(
                w_hbm.at[idx_v.at[g]], rows_v.at[b], gsem.at[b]
            ).wait()

        def write_copy(g, b):
            return pltpu.make_async_copy(
                rows_v.at[b],
                out_hbm.at[pl.ds(o0 + g * _GSZ, _GSZ)],
                wsem.at[b],
            )

        for b in range(_NBUF):
            start_gather(b, b)

        n_outer = (n_groups + _NBUF - 1) // _NBUF

        def body(j, carry):
            for b in range(_NBUF):
                g = j * _NBUF + b

                @pl.when(g < n_groups)
                def _():
                    wait_gather(g, b)
                    write_copy(g, b).start()

                    @pl.when(g + _NBUF < n_groups)
                    def _():
                        write_copy(g, b).wait()
                        start_gather(g + _NBUF, b)

            return carry

        lax.fori_loop(0, n_outer, body, 0)
        for b in range(_NBUF):
            write_copy(0, b).wait()

    return k(xpad, weight)


def kernel(x, weight):
    b, f = x.shape
    dim = weight.shape[1]
    info = plsc.get_sparse_core_info()
    nc, ns = info.num_cores, info.num_subcores
    fpad = 32
    xp = jnp.concatenate([x, x[:, : fpad - f]], axis=1).reshape(
        b * fpad // 128, 128
    )
    out = _embed_sc(xp, weight, nc, ns)
    return out.reshape(b, fpad, dim)[:, :f, :]


# speedup vs baseline: 5.3353x; 1.3892x over previous
import functools

import jax
import jax.numpy as jnp
from jax import lax
from jax.experimental import pallas as pl
from jax.experimental.pallas import tpu as pltpu
from jax.experimental.pallas import tpu_sc as plsc

_NBUF = 4
_GROUP = 4


@functools.partial(jax.jit, static_argnums=(2, 3, 4))
def _embed_sc(xflat, wpad, nc, ns, n_fields):
    num, wdim = wpad.shape
    dim = wdim // 2
    nw = nc * ns
    lpw = xflat.shape[1]
    gsz = _GROUP * n_fields
    n_groups = lpw // gsz
    bpw = lpw // n_fields
    bsz = nw * bpw
    mesh = plsc.VectorSubcoreMesh(core_axis_name="c", subcore_axis_name="s")

    @functools.partial(
        pl.kernel,
        out_type=jax.ShapeDtypeStruct((bsz, 32, 128), jnp.float32),
        mesh=mesh,
        scratch_types=[
            pltpu.VMEM((lpw,), jnp.int32),
            pltpu.VMEM((_NBUF, gsz, wdim), jnp.float32),
            pltpu.SemaphoreType.DMA((_NBUF,)),
            pltpu.SemaphoreType.DMA((_NBUF,)),
        ],
        compiler_params=pltpu.CompilerParams(use_tc_tiling_on_sc=False),
    )
    def k(x_hbm, w_hbm, out_hbm, idx_v, rows_v, gsem, wsem):
        wid = lax.axis_index("s") * nc + lax.axis_index("c")
        b0 = wid * bpw
        pltpu.sync_copy(x_hbm.at[wid], idx_v)

        def start_gather(g, b):
            pltpu.make_async_copy(
                w_hbm.at[idx_v.at[pl.ds(g * gsz, gsz)]], rows_v.at[b], gsem.at[b]
            ).start()

        def wait_gather(g, b):
            pltpu.make_async_copy(
                w_hbm.at[idx_v.at[pl.ds(g * gsz, gsz)]], rows_v.at[b], gsem.at[b]
            ).wait()

        def write_copy(g, b, i):
            return pltpu.make_async_copy(
                rows_v.at[b, pl.ds(i * n_fields, n_fields), pl.ds(0, dim)],
                out_hbm.at[b0 + g * _GROUP + i, pl.ds(0, n_fields), pl.ds(0, dim)],
                wsem.at[b],
            )

        for b in range(_NBUF):
            start_gather(b, b)

        n_outer = (n_groups + _NBUF - 1) // _NBUF

        def body(j, carry):
            for b in range(_NBUF):
                g = j * _NBUF + b

                @pl.when(g < n_groups)
                def _():
                    wait_gather(g, b)
                    for i in range(_GROUP):
                        write_copy(g, b, i).start()

                    @pl.when(g + _NBUF < n_groups)
                    def _():
                        for i in range(_GROUP):
                            write_copy(g, b, i).wait()
                        start_gather(g + _NBUF, b)

            return carry

        lax.fori_loop(0, n_outer, body, 0)
        for b in range(_NBUF):
            for i in range(_GROUP):
                write_copy(0, b, i).wait()

    out_big = k(xflat, wpad)
    return out_big[:, :n_fields, :dim]


def kernel(x, weight):
    b, f = x.shape
    info = plsc.get_sparse_core_info()
    nc, ns = info.num_cores, info.num_subcores
    nw = nc * ns
    xflat = x.reshape(nw, (b // nw) * f)
    wpad = jnp.pad(weight, ((0, 0), (0, weight.shape[1])))
    return _embed_sc(xflat, wpad, nc, ns, f)
